# baseline (speedup 1.0000x reference)
import functools

import jax
import jax.numpy as jnp
from jax import lax
from jax.experimental import pallas as pl
from jax.experimental.pallas import tpu as pltpu
from jax.experimental.pallas import tpu_sc as plsc

NC, NS, L = 2, 16, 16
NW = NC * NS
ED = 10
NV = 25
B, S = 16384, 200
NJT = S // 8
NJT_A = 128 // 8
ST = S - 128
IT_PER_W = (B // 128) // NW
I_PER_W = IT_PER_W * 128


@jax.jit
def _sc_embed(seq, tail, tab_t):
    mesh = plsc.VectorSubcoreMesh(
        core_axis_name="c", subcore_axis_name="s", num_cores=NC, num_subcores=NS
    )

    @functools.partial(
        pl.kernel,
        out_type=jax.ShapeDtypeStruct((ED, S, B), jnp.float32),
        mesh=mesh,
        compiler_params=pltpu.CompilerParams(
            needs_layout_passes=False,
            disable_bounds_checks=True,
            use_tc_tiling_on_sc=True,
        ),
        scratch_types=[
            pltpu.VMEM((ED * NV,), jnp.float32),
            pltpu.VMEM((I_PER_W, 128), jnp.int32),
            pltpu.VMEM((I_PER_W * ST,), jnp.int32),
            pltpu.VMEM((8, I_PER_W), jnp.int32),
            pltpu.VMEM((8, I_PER_W), jnp.float32),
            pltpu.VMEM((8, I_PER_W), jnp.float32),
            pltpu.VMEM((8, I_PER_W), jnp.float32),
            pltpu.VMEM((8, I_PER_W), jnp.float32),
            pltpu.SemaphoreType.DMA,
            pltpu.SemaphoreType.DMA,
            pltpu.SemaphoreType.DMA,
            pltpu.SemaphoreType.DMA,
        ],
    )
    def run(seq_hbm, tail_hbm, tab_hbm, out_hbm, tab_v, slab_a, slab_b, seqt_v,
            buf00, buf01, buf10, buf11, sem00, sem01, sem10, sem11):
        wid = lax.axis_index("s") * NC + lax.axis_index("c")
        i0 = wid * I_PER_W
        pltpu.sync_copy(tab_hbm, tab_v)
        pltpu.sync_copy(seq_hbm.at[pl.ds(i0, I_PER_W), pl.ds(0, 128)], slab_a)
        pltpu.sync_copy(tail_hbm.at[pl.ds(i0 * ST, I_PER_W * ST)], slab_b)
        lane = lax.iota(jnp.int32, L)
        lane_t = lane * ST
        bufs = ((buf00, buf01), (buf10, buf11))
        sems = ((sem00, sem01), (sem10, sem11))

        def make_plane(gather_col, first_jt):
            def plane(jt, carry):
                jcol0 = jt * 8

                @plsc.parallel_loop(0, 8)
                def trow(js):
                    for v16 in range(I_PER_W // L):
                        seqt_v[js, pl.ds(v16 * L, L)] = gather_col(
                            jcol0 + js, v16 * L)

                for k in range(ED // 2):
                    d0, d1 = 2 * k, 2 * k + 1
                    (b0, b1), (s0, s1) = bufs[k % 2], sems[k % 2]
                    dst0 = out_hbm.at[d0, pl.ds(jcol0, 8), pl.ds(i0, I_PER_W)]
                    dst1 = out_hbm.at[d1, pl.ds(jcol0, 8), pl.ds(i0, I_PER_W)]

                    if first_jt == 0 and k < 2:
                        @pl.when(jt > 0)
                        def _():
                            pltpu.make_async_copy(b0, dst0, s0).wait()
                            pltpu.make_async_copy(b1, dst1, s1).wait()
                    else:
                        pltpu.make_async_copy(b0, dst0, s0).wait()
                        pltpu.make_async_copy(b1, dst1, s1).wait()

                    @plsc.parallel_loop(0, 8)
                    def row(js):
                        for v16 in range(I_PER_W // L):
                            sv = seqt_v[js, pl.ds(v16 * L, L)]
                            val0 = plsc.load_gather(tab_v, [sv + d0 * NV])
                            val1 = plsc.load_gather(tab_v, [sv + d1 * NV])
                            b0[js, pl.ds(v16 * L, L)] = val0
                            b1[js, pl.ds(v16 * L, L)] = val1

                    pltpu.async_copy(b0, dst0, s0)
                    pltpu.async_copy(b1, dst1, s1)
                return carry

            return plane

        def col_a(jcol, iloc):
            return plsc.load_gather(slab_a, [lane + iloc, lane * 0 + jcol])

        def col_b(jcol, iloc):
            return plsc.load_gather(slab_b, [lane_t + (iloc * ST + jcol - 128)])

        lax.fori_loop(0, NJT_A, make_plane(col_a, 0), 0)
        lax.fori_loop(NJT_A, NJT, make_plane(col_b, NJT_A), 0)
        last = out_hbm.at[ED - 1, pl.ds((NJT - 1) * 8, 8), pl.ds(i0, I_PER_W)]
        pltpu.make_async_copy(buf00, last, sem00).wait()
        pltpu.make_async_copy(buf01, last, sem01).wait()
        pltpu.make_async_copy(buf10, last, sem10).wait()
        pltpu.make_async_copy(buf11, last, sem11).wait()

    return run(seq, tail, tab_t)


def kernel(sequence, table):
    seq = sequence.astype(jnp.int32)
    tail = seq[:, 128:].reshape(-1)
    tab_t = table.astype(jnp.float32).T.reshape(-1)
    out_t = _sc_embed(seq, tail, tab_t)
    return jnp.transpose(out_t, (2, 1, 0))
